# baseline (speedup 1.0000x reference)
import functools

import jax
import jax.numpy as jnp
from jax import lax
from jax.experimental import pallas as pl
from jax.experimental.pallas import tpu as pltpu
from jax.experimental.pallas import tpu_sc as plsc

B = 16384
D = 128
BB = 2048
G = B // BB
NC = 2
NS = 16
NW = NC * NS
RPW = B // NW
CH = 128
NCH = RPW // CH
L = 16


def _dot(a, b):
    return lax.dot_general(a, b, (((1,), (0,)), ((), ())),
                           preferred_element_type=jnp.float32)


def _combo_body(ac_t_ref, bt_t_ref, rt_t_ref, at_t_ref,
                w2_ref, b2_ref, wo_ref, bo_ref, t96_ref):
    f32 = jnp.float32
    wo = wo_ref[...]
    p_ac = _dot(ac_t_ref[...], wo[0:32, :])
    p_bt = _dot(bt_t_ref[...], wo[32:64, :])
    p_rt = _dot(rt_t_ref[...], wo[64:80, :])
    p_at = _dot(at_t_ref[...], wo[80:96, :])
    c0 = _dot(b2_ref[...], wo[96:128, :]) + bo_ref[...]

    def sel(n, f):
        rows = lax.broadcasted_iota(jnp.int32, (96, n), 0)
        cols = lax.broadcasted_iota(jnp.int32, (96, n), 1)
        return (f(rows) == cols).astype(f32)

    t96 = _dot(sel(4, lambda r: r // 24), p_ac)
    t96 += _dot(sel(4, lambda r: (r // 6) % 4), p_bt)
    t96 += _dot(sel(2, lambda r: (r // 3) % 2), p_rt)
    t96 += _dot(sel(3, lambda r: r % 3), p_at)
    t96_ref[...] = t96 + c0


def _sc_gather(t96, ac, bt, rt, at):
    mesh = plsc.VectorSubcoreMesh(core_axis_name="c", subcore_axis_name="s")
    f32 = jnp.float32

    @functools.partial(
        pl.kernel, mesh=mesh,
        out_type=jax.ShapeDtypeStruct((B, D), f32),
        scratch_types=[pltpu.VMEM((RPW,), jnp.int32),
                       pltpu.VMEM((RPW,), jnp.int32),
                       pltpu.VMEM((RPW,), jnp.int32),
                       pltpu.VMEM((RPW,), jnp.int32),
                       pltpu.VMEM((RPW,), jnp.int32),
                       pltpu.VMEM((RPW, D), f32),
                       pltpu.SemaphoreType.DMA],
    )
    def k(t96_h, ac_h, bt_h, rt_h, at_h, out_h,
          ia_v, ib_v, ir_v, it_v, ic_v, acc_v, sem):
        wid = lax.axis_index("s") * NC + lax.axis_index("c")
        base = wid * RPW
        pltpu.sync_copy(ac_h.at[pl.ds(base, RPW)], ia_v)
        pltpu.sync_copy(bt_h.at[pl.ds(base, RPW)], ib_v)
        pltpu.sync_copy(rt_h.at[pl.ds(base, RPW)], ir_v)
        pltpu.sync_copy(at_h.at[pl.ds(base, RPW)], it_v)
        for j in range(RPW // L):
            s = pl.ds(j * L, L)
            c = ((ia_v[s] * 4 + ib_v[s]) * 2 + ir_v[s]) * 3 + it_v[s]
            ic_v[s] = c
        copies = []
        for j in range(NCH):
            sl = pl.ds(j * CH, CH)
            copies.append(pltpu.async_copy(
                t96_h.at[ic_v.at[sl]], acc_v.at[sl], sem))
        for c in copies:
            c.wait()
        pltpu.sync_copy(acc_v, out_h.at[pl.ds(base, RPW)])

    return k(t96, ac, bt, rt, at)


def _tc_body(emb_ref, x_ref, w1_ref, b1_ref, w2_ref, wo_ref, out_ref):
    wo = wo_ref[...]
    w2p = _dot(w2_ref[...], wo[96:128, :])
    h = jnp.maximum(_dot(x_ref[...], w1_ref[...]) + b1_ref[...], 0.0)
    out_ref[...] = emb_ref[...] + _dot(h, w2p)


@jax.jit
def kernel(asset_class, borrower_type, rate_type, amort_type,
           continuous_features, ac_table, bt_table, rt_table, at_table,
           W1, b1, W2, b2, Wo, bo):
    n_cont = continuous_features.shape[1]
    full = lambda shape: pl.BlockSpec(shape, lambda *_: tuple(0 for _ in shape))
    row = lambda w: pl.BlockSpec((BB, w), lambda i: (i, 0))

    t96 = pl.pallas_call(
        _combo_body,
        in_specs=[full((4, 32)), full((4, 32)), full((2, 16)), full((3, 16)),
                  full((64, 32)), full((1, 32)),
                  full((128, 128)), full((1, 128))],
        out_specs=full((96, D)),
        out_shape=jax.ShapeDtypeStruct((96, D), jnp.float32),
    )(ac_table, bt_table, rt_table, at_table,
      W2, b2.reshape(1, 32), Wo, bo.reshape(1, 128))

    emb = _sc_gather(t96, asset_class, borrower_type, rate_type, amort_type)

    out = pl.pallas_call(
        _tc_body,
        grid=(G,),
        in_specs=[row(D), row(n_cont),
                  full((n_cont, 64)), full((1, 64)),
                  full((64, 32)), full((128, 128))],
        out_specs=row(D),
        out_shape=jax.ShapeDtypeStruct((B, D), jnp.float32),
        compiler_params=pltpu.CompilerParams(
            dimension_semantics=("arbitrary",)),
    )(emb, continuous_features, W1, b1.reshape(1, 64), W2, Wo)
    return out
